# baseline (speedup 1.0000x reference)
import functools

import jax
import jax.numpy as jnp
from jax import lax
from jax.experimental import pallas as pl
from jax.experimental.pallas import tpu as pltpu
from jax.experimental.pallas import tpu_sc as plsc

B = 4096
F = 26
EMB = 4
VOCAB = 100000
NC = 2
NS = 16
NW = NC * NS
PB = B // NW
FB = F * PB
L = 16
GROUPS = PB // L
RS_ROWS = VOCAB * EMB // 128
RS_COLS = 128 // EMB


def _prep_body(x_ref, fids_ref, rs_ref, idxt_ref):
    x = x_ref[...]
    ii = lax.broadcasted_iota(jnp.int32, (128, RS_COLS), 0)
    jj = lax.broadcasted_iota(jnp.int32, (128, RS_COLS), 1)
    m = jnp.where(ii // EMB == jj, 1.0, 0.0).astype(jnp.float32)
    rs_ref[...] = jnp.dot(x, m, preferred_element_type=jnp.float32)
    idxt_ref[...] = fids_ref[...].T


_prep = pl.pallas_call(
    _prep_body,
    out_shape=(
        jax.ShapeDtypeStruct((RS_ROWS, RS_COLS), jnp.float32),
        jax.ShapeDtypeStruct((F, B), jnp.int32),
    ),
)


def _sc_body(idxt_hbm, rowsum_hbm, out_hbm, idx_v, vals_v, out_v, sem):
    c = lax.axis_index("c")
    s = lax.axis_index("s")
    wid = s * NC + c
    base = wid * PB

    idx_cps = [
        pltpu.async_copy(
            idxt_hbm.at[f, pl.ds(base, PB)], idx_v.at[pl.ds(f * PB, PB)], sem
        )
        for f in range(F)
    ]
    for cp in idx_cps:
        cp.wait()

    copies = [
        pltpu.async_copy(
            rowsum_hbm.at[idx_v.at[pl.ds(f * PB, PB)]],
            vals_v.at[pl.ds(f * PB, PB)],
            sem,
        )
        for f in range(F)
    ]
    for cp in copies:
        cp.wait()

    def group(j, carry):
        b0 = j * L
        acc = vals_v[pl.ds(b0, L)]
        for f in range(1, F):
            acc = acc + vals_v[pl.ds(f * PB + b0, L)]
        out_v[pl.ds(b0, L)] = acc
        return carry

    lax.fori_loop(0, GROUPS, group, 0)

    pltpu.sync_copy(out_v, out_hbm.at[pl.ds(wid * PB, PB)])


@functools.partial(
    pl.kernel,
    out_type=jax.ShapeDtypeStruct((B,), jnp.float32),
    mesh=plsc.VectorSubcoreMesh(
        core_axis_name="c", subcore_axis_name="s", num_cores=NC, num_subcores=NS
    ),
    scratch_types=[
        pltpu.VMEM((FB,), jnp.int32),
        pltpu.VMEM((FB,), jnp.float32),
        pltpu.VMEM((PB,), jnp.float32),
        pltpu.SemaphoreType.DMA,
    ],
    compiler_params=pltpu.CompilerParams(use_tc_tiling_on_sc=False),
)
def _sc_kernel(idxt_hbm, rowsum_hbm, out_hbm, idx_v, vals_v, out_v, sem):
    _sc_body(idxt_hbm, rowsum_hbm, out_hbm, idx_v, vals_v, out_v, sem)


def kernel(fids_batch, table):
    rs2d, idxt = _prep(
        table.reshape(RS_ROWS, 128), fids_batch.astype(jnp.int32)
    )
    return _sc_kernel(idxt, rs2d.reshape(VOCAB))


# speedup vs baseline: 3.2720x; 3.2720x over previous
import functools

import jax
import jax.numpy as jnp
from jax import lax
from jax.experimental import pallas as pl
from jax.experimental.pallas import tpu as pltpu
from jax.experimental.pallas import tpu_sc as plsc

B = 4096
F = 26
EMB = 4
VOCAB = 100000
NC = 2
NS = 16
NW = NC * NS
PB = B // NW
FB = F * PB
L = 16
GROUPS = PB // L
CH = 6256
VP = NS * CH


def _sc_body(idxt_hbm, tabt_hbm, out_hbm, rs_hbm, cbuf_v, rs_v, idx_v,
             vals_v, out_v, sem):
    c = lax.axis_index("c")
    s = lax.axis_index("s")
    wid = s * NC + c

    r0 = s * CH
    col_cps = [
        pltpu.async_copy(
            tabt_hbm.at[e, pl.ds(r0, CH)], cbuf_v.at[pl.ds(e * CH, CH)], sem
        )
        for e in range(EMB)
    ]
    for cp in col_cps:
        cp.wait()

    def rs_group(k, carry):
        o = k * L
        acc = cbuf_v[pl.ds(o, L)]
        for e in range(1, EMB):
            acc = acc + cbuf_v[pl.ds(e * CH + o, L)]
        rs_v[pl.ds(o, L)] = acc
        return carry

    lax.fori_loop(0, CH // L, rs_group, 0)
    pltpu.sync_copy(rs_v, rs_hbm.at[c, pl.ds(r0, CH)])
    plsc.subcore_barrier()

    pltpu.sync_copy(idxt_hbm.at[pl.ds(wid * FB, FB)], idx_v)
    my_rs = rs_hbm.at[c]
    gat_cps = [
        pltpu.async_copy(
            my_rs.at[idx_v.at[pl.ds(f * PB, PB)]],
            vals_v.at[pl.ds(f * PB, PB)],
            sem,
        )
        for f in range(F)
    ]
    for cp in gat_cps:
        cp.wait()

    def group(j, carry):
        b0 = j * L
        acc = vals_v[pl.ds(b0, L)]
        for f in range(1, F):
            acc = acc + vals_v[pl.ds(f * PB + b0, L)]
        out_v[pl.ds(b0, L)] = acc
        return carry

    lax.fori_loop(0, GROUPS, group, 0)
    pltpu.sync_copy(out_v, out_hbm.at[pl.ds(wid * PB, PB)])


@functools.partial(
    pl.kernel,
    out_type=(
        jax.ShapeDtypeStruct((B,), jnp.float32),
        jax.ShapeDtypeStruct((NC, VP), jnp.float32),
    ),
    mesh=plsc.VectorSubcoreMesh(
        core_axis_name="c", subcore_axis_name="s", num_cores=NC, num_subcores=NS
    ),
    scratch_types=[
        pltpu.VMEM((EMB * CH,), jnp.float32),
        pltpu.VMEM((CH,), jnp.float32),
        pltpu.VMEM((FB,), jnp.int32),
        pltpu.VMEM((FB,), jnp.float32),
        pltpu.VMEM((PB,), jnp.float32),
        pltpu.SemaphoreType.DMA,
    ],
    compiler_params=pltpu.CompilerParams(use_tc_tiling_on_sc=False),
)
def _sc_kernel(idxt_hbm, tabt_hbm, out_hbm, rs_hbm, cbuf_v, rs_v, idx_v,
               vals_v, out_v, sem):
    _sc_body(idxt_hbm, tabt_hbm, out_hbm, rs_hbm, cbuf_v, rs_v, idx_v,
             vals_v, out_v, sem)


def kernel(fids_batch, table):
    idxt = (
        fids_batch.astype(jnp.int32)
        .T.reshape(F, NW, PB)
        .transpose(1, 0, 2)
        .reshape(-1)
    )
    tabt = jnp.pad(table.T, ((0, 0), (0, VP - VOCAB)))
    out, _ = _sc_kernel(idxt, tabt)
    return out


# speedup vs baseline: 3.4369x; 1.0504x over previous
import functools

import jax
import jax.numpy as jnp
from jax import lax
from jax.experimental import pallas as pl
from jax.experimental.pallas import tpu as pltpu
from jax.experimental.pallas import tpu_sc as plsc

B = 4096
F = 26
EMB = 4
VOCAB = 100000
NC = 2
NS = 16
NW = NC * NS
PB = B // NW
FB = F * PB
L = 16
GROUPS = PB // L
CH = 6272
VP = NS * CH


def _sc_body(idxt_hbm, tabt_hbm, out_hbm, rs_hbm, cbuf_v, rs_v, idx_v,
             vals_v, out_v, sem, sem2):
    c = lax.axis_index("c")
    s = lax.axis_index("s")
    wid = s * NC + c

    idx_cp = pltpu.async_copy(
        idxt_hbm.at[pl.ds(wid * FB, FB)], idx_v, sem2
    )

    r0 = s * CH
    col_cps = [
        pltpu.async_copy(
            tabt_hbm.at[e, pl.ds(r0, CH)], cbuf_v.at[pl.ds(e * CH, CH)], sem
        )
        for e in range(EMB)
    ]
    for cp in col_cps:
        cp.wait()

    UNROLL = 8

    def rs_group(k, carry):
        for u in range(UNROLL):
            o = (k * UNROLL + u) * L
            acc = cbuf_v[pl.ds(o, L)]
            for e in range(1, EMB):
                acc = acc + cbuf_v[pl.ds(e * CH + o, L)]
            rs_v[pl.ds(o, L)] = acc
        return carry

    lax.fori_loop(0, CH // L // UNROLL, rs_group, 0)
    pltpu.sync_copy(rs_v, rs_hbm.at[c, pl.ds(r0, CH)])
    plsc.subcore_barrier()

    idx_cp.wait()
    my_rs = rs_hbm.at[c]
    gat_cps = [
        pltpu.async_copy(
            my_rs.at[idx_v.at[pl.ds(f * PB, PB)]],
            vals_v.at[pl.ds(f * PB, PB)],
            sem,
        )
        for f in range(F)
    ]
    for cp in gat_cps:
        cp.wait()

    def group(j, carry):
        b0 = j * L
        acc = vals_v[pl.ds(b0, L)]
        for f in range(1, F):
            acc = acc + vals_v[pl.ds(f * PB + b0, L)]
        out_v[pl.ds(b0, L)] = acc
        return carry

    lax.fori_loop(0, GROUPS, group, 0)
    pltpu.sync_copy(out_v, out_hbm.at[pl.ds(wid * PB, PB)])


@functools.partial(
    pl.kernel,
    out_type=(
        jax.ShapeDtypeStruct((B,), jnp.float32),
        jax.ShapeDtypeStruct((NC, VP), jnp.float32),
    ),
    mesh=plsc.VectorSubcoreMesh(
        core_axis_name="c", subcore_axis_name="s", num_cores=NC, num_subcores=NS
    ),
    scratch_types=[
        pltpu.VMEM((EMB * CH,), jnp.float32),
        pltpu.VMEM((CH,), jnp.float32),
        pltpu.VMEM((FB,), jnp.int32),
        pltpu.VMEM((FB,), jnp.float32),
        pltpu.VMEM((PB,), jnp.float32),
        pltpu.SemaphoreType.DMA,
        pltpu.SemaphoreType.DMA,
    ],
    compiler_params=pltpu.CompilerParams(use_tc_tiling_on_sc=False),
)
def _sc_kernel(idxt_hbm, tabt_hbm, out_hbm, rs_hbm, cbuf_v, rs_v, idx_v,
               vals_v, out_v, sem, sem2):
    _sc_body(idxt_hbm, tabt_hbm, out_hbm, rs_hbm, cbuf_v, rs_v, idx_v,
             vals_v, out_v, sem, sem2)


def kernel(fids_batch, table):
    idxt = (
        fids_batch.astype(jnp.int32)
        .T.reshape(F, NW, PB)
        .transpose(1, 0, 2)
        .reshape(-1)
    )
    tabt = jnp.pad(table.T, ((0, 0), (0, VP - VOCAB)))
    out, _ = _sc_kernel(idxt, tabt)
    return out


# speedup vs baseline: 3.9530x; 1.1502x over previous
import functools

import jax
import jax.numpy as jnp
from jax import lax
from jax.experimental import pallas as pl
from jax.experimental.pallas import tpu as pltpu
from jax.experimental.pallas import tpu_sc as plsc

B = 4096
F = 26
EMB = 4
VOCAB = 100000
NC = 2
NS = 16
NW = NC * NS
PB = B // NW
FB = F * PB
L = 16
GROUPS = PB // L
CH = 6272
VP = NS * CH


def _sc_body(idxt_hbm, tabt_hbm, out_hbm, rs_sh, cbuf_v, rs_v, idx_v,
             vals_v, out_v, sem, sem2):
    c = lax.axis_index("c")
    s = lax.axis_index("s")
    wid = s * NC + c

    idx_cp = pltpu.async_copy(
        idxt_hbm.at[pl.ds(wid * FB, FB)], idx_v, sem2
    )

    r0 = s * CH
    col_cps = [
        pltpu.async_copy(
            tabt_hbm.at[e, pl.ds(r0, CH)], cbuf_v.at[pl.ds(e * CH, CH)], sem
        )
        for e in range(EMB)
    ]
    for cp in col_cps:
        cp.wait()

    UNROLL = 8

    def rs_group(k, carry):
        for u in range(UNROLL):
            o = (k * UNROLL + u) * L
            acc = cbuf_v[pl.ds(o, L)]
            for e in range(1, EMB):
                acc = acc + cbuf_v[pl.ds(e * CH + o, L)]
            rs_v[pl.ds(o, L)] = acc
        return carry

    lax.fori_loop(0, CH // L // UNROLL, rs_group, 0)
    pltpu.sync_copy(rs_v, rs_sh.at[pl.ds(r0, CH)])
    plsc.subcore_barrier()

    idx_cp.wait()
    gat_cps = [
        pltpu.async_copy(
            rs_sh.at[idx_v.at[pl.ds(f * PB, PB)]],
            vals_v.at[pl.ds(f * PB, PB)],
            sem,
        )
        for f in range(F)
    ]
    for cp in gat_cps:
        cp.wait()

    def group(j, carry):
        b0 = j * L
        acc = vals_v[pl.ds(b0, L)]
        for f in range(1, F):
            acc = acc + vals_v[pl.ds(f * PB + b0, L)]
        out_v[pl.ds(b0, L)] = acc
        return carry

    lax.fori_loop(0, GROUPS, group, 0)
    pltpu.sync_copy(out_v, out_hbm.at[pl.ds(wid * PB, PB)])


@functools.partial(
    pl.kernel,
    out_type=jax.ShapeDtypeStruct((B,), jnp.float32),
    mesh=plsc.VectorSubcoreMesh(
        core_axis_name="c", subcore_axis_name="s", num_cores=NC, num_subcores=NS
    ),
    scratch_types=[
        pltpu.VMEM_SHARED((VP,), jnp.float32),
        pltpu.VMEM((EMB * CH,), jnp.float32),
        pltpu.VMEM((CH,), jnp.float32),
        pltpu.VMEM((FB,), jnp.int32),
        pltpu.VMEM((FB,), jnp.float32),
        pltpu.VMEM((PB,), jnp.float32),
        pltpu.SemaphoreType.DMA,
        pltpu.SemaphoreType.DMA,
    ],
    compiler_params=pltpu.CompilerParams(use_tc_tiling_on_sc=False),
)
def _sc_kernel(idxt_hbm, tabt_hbm, out_hbm, rs_sh, cbuf_v, rs_v, idx_v,
               vals_v, out_v, sem, sem2):
    _sc_body(idxt_hbm, tabt_hbm, out_hbm, rs_sh, cbuf_v, rs_v, idx_v,
             vals_v, out_v, sem, sem2)


def kernel(fids_batch, table):
    idxt = (
        fids_batch.astype(jnp.int32)
        .T.reshape(F, NW, PB)
        .transpose(1, 0, 2)
        .reshape(-1)
    )
    tabt = jnp.pad(table.T, ((0, 0), (0, VP - VOCAB)))
    return _sc_kernel(idxt, tabt)
